# baseline (speedup 1.0000x reference)
import functools

import jax
import jax.numpy as jnp
from jax import lax
from jax.experimental import pallas as pl
from jax.experimental.pallas import tpu as pltpu
from jax.experimental.pallas import tpu_sc as plsc

G = 1024
N_NODES = 100000
BN = 10000
NUM_BLOCKS = N_NODES // (2 * BN)

NC = 2
NS = 16
NW = NC * NS
CHUNK = 128
NCHUNK = 25
BW = CHUNK * NCHUNK
NPAD = BW * NW


def _mlp_body(xa_ref, xb_ref, w1_ref, b1_ref, w2_ref, b2_ref, out_ref):
    for k, x_ref in enumerate((xa_ref, xb_ref)):
        x = x_ref[...]
        h = jnp.dot(x, w1_ref[...], preferred_element_type=jnp.float32)
        h = h + b1_ref[...]
        h = 0.5 * h * (1.0 + lax.tanh(0.5 * h))
        e = lax.dot_general(w2_ref[...], h, (((0,), (1,)), ((), ())),
                            preferred_element_type=jnp.float32)
        out_ref[:, k] = e + b2_ref[...]


_mesh = plsc.VectorSubcoreMesh(core_axis_name="c", subcore_axis_name="s")


@functools.partial(
    pl.kernel,
    mesh=_mesh,
    out_type=jax.ShapeDtypeStruct((NC, G), jnp.float32),
    scratch_types=[
        pltpu.VMEM((NCHUNK, CHUNK), jnp.float32),
        pltpu.VMEM((NCHUNK, CHUNK), jnp.int32),
        pltpu.VMEM_SHARED((G,), jnp.float32),
        pltpu.SemaphoreType.DMA,
    ],
)
def _segsum(e_hbm, idx_hbm, zeros_hbm, out_hbm, e_v, idx_v, acc_sh, sem):
    c = lax.axis_index("c")
    s = lax.axis_index("s")
    wid = s * NC + c

    pltpu.sync_copy(e_hbm.at[wid], e_v)
    pltpu.sync_copy(idx_hbm.at[wid], idx_v)

    @pl.when(s == 0)
    def _():
        pltpu.sync_copy(zeros_hbm, acc_sh)

    plsc.subcore_barrier()

    def fire(j, carry):
        pltpu.async_copy(e_v.at[j], acc_sh.at[idx_v.at[j]], sem, add=True)
        return carry

    lax.fori_loop(0, NCHUNK, fire, 0)

    def drain(j, carry):
        pltpu.make_async_copy(e_v.at[j], acc_sh.at[idx_v.at[j]], sem).wait()
        return carry

    lax.fori_loop(0, NCHUNK, drain, 0)

    plsc.subcore_barrier()

    @pl.when(s == 0)
    def _():
        pltpu.sync_copy(acc_sh, out_hbm.at[c])


def kernel(node_scalar, batch, W1, b1, W2, b2):
    n, d = node_scalar.shape
    hdim = W1.shape[1]
    assert n == N_NODES

    e = pl.pallas_call(
        _mlp_body,
        grid=(NUM_BLOCKS,),
        in_specs=[
            pl.BlockSpec((BN, d), lambda i: (i, 0)),
            pl.BlockSpec((BN, d), lambda i: (i + NUM_BLOCKS, 0)),
            pl.BlockSpec((d, hdim), lambda i: (0, 0)),
            pl.BlockSpec((1, hdim), lambda i: (0, 0)),
            pl.BlockSpec((hdim, 1), lambda i: (0, 0)),
            pl.BlockSpec((1, 1), lambda i: (0, 0)),
        ],
        out_specs=pl.BlockSpec((1, 2, BN), lambda i: (i, 0, 0)),
        out_shape=jax.ShapeDtypeStruct((NUM_BLOCKS, 2, BN), jnp.float32),
    )(node_scalar, node_scalar, W1, b1.reshape(1, hdim), W2,
      b2.reshape(1, 1))

    e_pad = jnp.concatenate(
        [e[:, 0].reshape(n // 2), e[:, 1].reshape(n // 2),
         jnp.zeros((NPAD - n,), jnp.float32)])
    ids_pad = jnp.concatenate(
        [batch.astype(jnp.int32), jnp.zeros((NPAD - n,), jnp.int32)])
    e3 = e_pad.reshape(NW, NCHUNK, CHUNK)
    idx3 = ids_pad.reshape(NW, NCHUNK, CHUNK)

    partials = _segsum(e3, idx3, jnp.zeros((G,), jnp.float32))
    return partials[0] + partials[1]


# speedup vs baseline: 1.0684x; 1.0684x over previous
import functools

import jax
import jax.numpy as jnp
from jax import lax
from jax.experimental import pallas as pl
from jax.experimental.pallas import tpu as pltpu
from jax.experimental.pallas import tpu_sc as plsc

G = 1024
N_NODES = 100000
BN = 20000
NUM_BLOCKS = N_NODES // BN

NC = 2
NS = 16
NW = NC * NS
CHUNK = 128
NCHUNK = 25
BW = CHUNK * NCHUNK
NPAD = BW * NW


def _mlp_body(x_ref, w1_ref, b1_ref, w2_ref, b2_ref, out_ref):
    x = x_ref[...]
    h = jnp.dot(x, w1_ref[...], preferred_element_type=jnp.float32)
    h = h + b1_ref[...]
    h = 0.5 * h * (1.0 + lax.tanh(0.5 * h))
    e = lax.dot_general(w2_ref[...], h, (((0,), (1,)), ((), ())),
                        preferred_element_type=jnp.float32)
    out_ref[0] = e + b2_ref[...]


_mesh = plsc.VectorSubcoreMesh(core_axis_name="c", subcore_axis_name="s")


@functools.partial(
    pl.kernel,
    mesh=_mesh,
    out_type=jax.ShapeDtypeStruct((NC, G), jnp.float32),
    scratch_types=[
        pltpu.VMEM((NCHUNK, CHUNK), jnp.float32),
        pltpu.VMEM((NCHUNK, CHUNK), jnp.int32),
        pltpu.VMEM_SHARED((G,), jnp.float32),
        pltpu.SemaphoreType.DMA,
    ],
)
def _segsum(e_hbm, idx_hbm, zeros_hbm, out_hbm, e_v, idx_v, acc_sh, sem):
    c = lax.axis_index("c")
    s = lax.axis_index("s")
    wid = s * NC + c

    pltpu.async_copy(e_hbm.at[wid], e_v, sem)
    pltpu.async_copy(idx_hbm.at[wid], idx_v, sem)

    @pl.when(s == 0)
    def _():
        pltpu.sync_copy(zeros_hbm, acc_sh)

    pltpu.make_async_copy(e_hbm.at[wid], e_v, sem).wait()
    pltpu.make_async_copy(idx_hbm.at[wid], idx_v, sem).wait()

    plsc.subcore_barrier()

    def fire(j, carry):
        pltpu.async_copy(e_v.at[j], acc_sh.at[idx_v.at[j]], sem, add=True)
        return carry

    lax.fori_loop(0, NCHUNK, fire, 0)

    def drain(j, carry):
        pltpu.make_async_copy(e_v.at[j], acc_sh.at[idx_v.at[j]], sem).wait()
        return carry

    lax.fori_loop(0, NCHUNK, drain, 0)

    plsc.subcore_barrier()

    @pl.when(s == 0)
    def _():
        pltpu.sync_copy(acc_sh, out_hbm.at[c])


def kernel(node_scalar, batch, W1, b1, W2, b2):
    n, d = node_scalar.shape
    hdim = W1.shape[1]
    assert n == N_NODES

    e = pl.pallas_call(
        _mlp_body,
        grid=(NUM_BLOCKS,),
        in_specs=[
            pl.BlockSpec((BN, d), lambda i: (i, 0)),
            pl.BlockSpec((d, hdim), lambda i: (0, 0)),
            pl.BlockSpec((1, hdim), lambda i: (0, 0)),
            pl.BlockSpec((hdim, 1), lambda i: (0, 0)),
            pl.BlockSpec((1, 1), lambda i: (0, 0)),
        ],
        out_specs=pl.BlockSpec((1, 1, BN), lambda i: (i, 0, 0)),
        out_shape=jax.ShapeDtypeStruct((NUM_BLOCKS, 1, BN), jnp.float32),
    )(node_scalar, W1, b1.reshape(1, hdim), W2, b2.reshape(1, 1))

    e_pad = jnp.concatenate(
        [e.reshape(n), jnp.zeros((NPAD - n,), jnp.float32)])
    ids_pad = jnp.concatenate(
        [batch.astype(jnp.int32), jnp.zeros((NPAD - n,), jnp.int32)])
    e3 = e_pad.reshape(NW, NCHUNK, CHUNK)
    idx3 = ids_pad.reshape(NW, NCHUNK, CHUNK)

    partials = _segsum(e3, idx3, jnp.zeros((G,), jnp.float32))
    return partials[0] + partials[1]
